# baseline (speedup 1.0000x reference)
import functools

import jax
import jax.numpy as jnp
from jax import lax
from jax.experimental import pallas as pl
from jax.experimental.pallas import tpu as pltpu
from jax.experimental.pallas import tpu_sc as plsc

_NUM_CORES = 2
_NUM_SUBCORES = 16
_NW = _NUM_CORES * _NUM_SUBCORES
_CHUNK = 2000


@functools.partial(jax.jit, static_argnames=("n_edges", "dim"))
def _sc_embedding_lookup(table, idx_flat, *, n_edges, dim):
    b_per_w = n_edges // _NW
    n_chunks = b_per_w // _CHUNK
    mesh = plsc.VectorSubcoreMesh(core_axis_name="c", subcore_axis_name="s")

    @functools.partial(
        pl.kernel,
        mesh=mesh,
        out_type=jax.ShapeDtypeStruct((n_edges, dim), jnp.float32),
        scratch_types=[
            pltpu.VMEM((_CHUNK,), jnp.int32),
            pltpu.VMEM((_CHUNK, dim), jnp.float32),
            pltpu.SemaphoreType.DMA,
        ],
        compiler_params=pltpu.CompilerParams(use_tc_tiling_on_sc=False),
    )
    def k(table_hbm, idx_hbm, out_hbm, idx_v, rows_v, gsem):
        wid = lax.axis_index("s") * _NUM_CORES + lax.axis_index("c")
        w_base = wid * b_per_w

        def body(c, _):
            base = w_base + c * _CHUNK
            pltpu.sync_copy(idx_hbm.at[pl.ds(base, _CHUNK)], idx_v)
            pltpu.async_copy(table_hbm.at[idx_v], rows_v, gsem).wait()
            pltpu.sync_copy(rows_v, out_hbm.at[pl.ds(base, _CHUNK)])
            return ()

        lax.fori_loop(0, n_chunks, body, ())

    return k(table, idx_flat)


def kernel(edge_types, table):
    batch, n_edges = edge_types.shape
    dim = table.shape[1]
    idx_flat = edge_types.reshape(n_edges).astype(jnp.int32)
    out = _sc_embedding_lookup(table, idx_flat, n_edges=n_edges, dim=dim)
    return out.reshape(batch, n_edges, dim)


# speedup vs baseline: 7.2215x; 7.2215x over previous
import functools

import jax
import jax.numpy as jnp
from jax import lax
from jax.experimental import pallas as pl
from jax.experimental.pallas import tpu as pltpu
from jax.experimental.pallas import tpu_sc as plsc

_NUM_CORES = 2
_NUM_SUBCORES = 16
_NW = _NUM_CORES * _NUM_SUBCORES
_CHUNK = 2560
_LANES = 16


@functools.partial(jax.jit, static_argnames=("n_edges", "n_types", "dim"))
def _sc_embedding_lookup(table, edge_types, *, n_edges, n_types, dim):
    total_chunks = n_edges // _CHUNK
    assert total_chunks * _CHUNK == n_edges
    base_cnt = total_chunks // _NW
    rem = total_chunks % _NW
    max_cnt = base_cnt + (1 if rem else 0)
    max_cnt += max_cnt % 2
    mesh = plsc.VectorSubcoreMesh(core_axis_name="c", subcore_axis_name="s")

    @functools.partial(
        pl.kernel,
        mesh=mesh,
        out_type=jax.ShapeDtypeStruct((1, n_edges, dim), jnp.float32),
        scratch_types=[
            pltpu.VMEM((n_types * dim,), jnp.float32),
            pltpu.VMEM((_CHUNK,), jnp.int32),
            pltpu.VMEM((_CHUNK,), jnp.int32),
            pltpu.VMEM((_CHUNK, dim), jnp.float32),
            pltpu.VMEM((_CHUNK, dim), jnp.float32),
            pltpu.SemaphoreType.DMA,
            pltpu.SemaphoreType.DMA,
            pltpu.SemaphoreType.DMA,
        ],
        compiler_params=pltpu.CompilerParams(
            needs_layout_passes=False, use_tc_tiling_on_sc=False),
    )
    def k(table_hbm, idx_hbm, out_hbm, table_v, idx0, idx1, rows0, rows1,
          isem, ssem0, ssem1):
        wid = lax.axis_index("s") * _NUM_CORES + lax.axis_index("c")
        n_w = base_cnt + jnp.where(wid < rem, 1, 0)
        start_w = wid * base_cnt + jnp.minimum(wid, rem)
        idx_bufs = (idx0, idx1)
        rows_bufs = (rows0, rows1)
        ssems = (ssem0, ssem1)

        pltpu.sync_copy(table_hbm, table_v)
        pltpu.async_copy(
            idx_hbm.at[0, pl.ds(start_w * _CHUNK, _CHUNK)], idx0, isem)

        lane = lax.iota(jnp.int32, _LANES)

        def compute_chunk(idx_v, rows_v):
            def grp(g, _):
                ivec = idx_v[pl.ds(g * _LANES, _LANES)] * dim
                row_ix = lane + g * _LANES
                for d in range(dim):
                    col_ix = jnp.full((_LANES,), d, jnp.int32)
                    vals = plsc.load_gather(table_v, [ivec + d])
                    plsc.store_scatter(rows_v, [row_ix, col_ix], vals)
                return ()

            lax.fori_loop(0, _CHUNK // _LANES, grp, (), unroll=False)

        def outer(o, _):
            for b in range(2):
                i = o * 2 + b

                @pl.when(i < n_w)
                def _():
                    base = (start_w + i) * _CHUNK
                    pltpu.make_async_copy(
                        idx_hbm.at[0, pl.ds(base, _CHUNK)], idx_bufs[b], isem
                    ).wait()

                    @pl.when(i + 1 < n_w)
                    def _():
                        pltpu.async_copy(
                            idx_hbm.at[0, pl.ds(base + _CHUNK, _CHUNK)],
                            idx_bufs[1 - b], isem)

                    @pl.when(i >= 2)
                    def _():
                        pltpu.make_async_copy(
                            rows_bufs[b],
                            out_hbm.at[0, pl.ds(base, _CHUNK)],
                            ssems[b]).wait()

                    compute_chunk(idx_bufs[b], rows_bufs[b])
                    pltpu.async_copy(
                        rows_bufs[b], out_hbm.at[0, pl.ds(base, _CHUNK)],
                        ssems[b])
            return ()

        lax.fori_loop(0, max_cnt // 2, outer, ())

        for b in range(2):
            pltpu.make_async_copy(
                rows_bufs[b], out_hbm.at[0, pl.ds(0, _CHUNK)],
                ssems[b]).wait()

    return k(table, edge_types)


def kernel(edge_types, table):
    batch, n_edges = edge_types.shape
    n_types, dim = table.shape
    out = _sc_embedding_lookup(
        table.reshape(-1), edge_types.astype(jnp.int32),
        n_edges=n_edges, n_types=n_types, dim=dim)
    return out


# speedup vs baseline: 8.9618x; 1.2410x over previous
import functools

import jax
import jax.numpy as jnp
from jax import lax
from jax.experimental import pallas as pl
from jax.experimental.pallas import tpu as pltpu
from jax.experimental.pallas import tpu_sc as plsc

_NUM_CORES = 2
_NUM_SUBCORES = 16
_NW = _NUM_CORES * _NUM_SUBCORES
_CHUNK = 2560
_LANES = 16


@functools.partial(jax.jit, static_argnames=("n_edges", "n_types", "dim"))
def _sc_embedding_lookup(table, edge_types, *, n_edges, n_types, dim):
    total_chunks = n_edges // _CHUNK
    assert total_chunks * _CHUNK == n_edges
    base_cnt = total_chunks // _NW
    rem = total_chunks % _NW
    max_cnt = base_cnt + (1 if rem else 0)
    max_cnt += max_cnt % 2
    mesh = plsc.VectorSubcoreMesh(core_axis_name="c", subcore_axis_name="s")

    @functools.partial(
        pl.kernel,
        mesh=mesh,
        out_type=jax.ShapeDtypeStruct((1, n_edges, dim), jnp.float32),
        scratch_types=[
            pltpu.VMEM((n_types * dim,), jnp.float32),
            pltpu.VMEM((_CHUNK,), jnp.int32),
            pltpu.VMEM((_CHUNK,), jnp.int32),
            pltpu.VMEM((_CHUNK, dim), jnp.float32),
            pltpu.VMEM((_CHUNK, dim), jnp.float32),
            pltpu.SemaphoreType.DMA,
            pltpu.SemaphoreType.DMA,
            pltpu.SemaphoreType.DMA,
        ],
        compiler_params=pltpu.CompilerParams(
            needs_layout_passes=False, use_tc_tiling_on_sc=False),
    )
    def k(table_hbm, idx_hbm, out_hbm, table_v, idx0, idx1, rows0, rows1,
          isem, ssem0, ssem1):
        wid = lax.axis_index("s") * _NUM_CORES + lax.axis_index("c")
        n_w = base_cnt + jnp.where(wid < rem, 1, 0)
        start_w = wid * base_cnt + jnp.minimum(wid, rem)
        idx_bufs = (idx0, idx1)
        rows_bufs = (rows0, rows1)
        ssems = (ssem0, ssem1)

        pltpu.sync_copy(table_hbm, table_v)
        pltpu.async_copy(
            idx_hbm.at[0, pl.ds(start_w * _CHUNK, _CHUNK)], idx0, isem)

        def compute_chunk(idx_v, rows_v):
            def grp(g, _):
                e0 = g * _LANES
                ivec = idx_v[pl.ds(e0, _LANES)] * dim
                for u in range(_LANES):
                    rows_v[e0 + u] = table_v[pl.ds(ivec[u], dim)]
                return ()

            lax.fori_loop(0, _CHUNK // _LANES, grp, (), unroll=False)

        def outer(o, _):
            for b in range(2):
                i = o * 2 + b

                @pl.when(i < n_w)
                def _():
                    base = (start_w + i) * _CHUNK
                    pltpu.make_async_copy(
                        idx_hbm.at[0, pl.ds(base, _CHUNK)], idx_bufs[b], isem
                    ).wait()

                    @pl.when(i + 1 < n_w)
                    def _():
                        pltpu.async_copy(
                            idx_hbm.at[0, pl.ds(base + _CHUNK, _CHUNK)],
                            idx_bufs[1 - b], isem)

                    @pl.when(i >= 2)
                    def _():
                        pltpu.make_async_copy(
                            rows_bufs[b],
                            out_hbm.at[0, pl.ds(base, _CHUNK)],
                            ssems[b]).wait()

                    compute_chunk(idx_bufs[b], rows_bufs[b])
                    pltpu.async_copy(
                        rows_bufs[b], out_hbm.at[0, pl.ds(base, _CHUNK)],
                        ssems[b])
            return ()

        lax.fori_loop(0, max_cnt // 2, outer, ())

        for b in range(2):
            pltpu.make_async_copy(
                rows_bufs[b], out_hbm.at[0, pl.ds(0, _CHUNK)],
                ssems[b]).wait()

    return k(table, edge_types)


def kernel(edge_types, table):
    batch, n_edges = edge_types.shape
    n_types, dim = table.shape
    out = _sc_embedding_lookup(
        table.reshape(-1), edge_types.astype(jnp.int32),
        n_edges=n_edges, n_types=n_types, dim=dim)
    return out


# speedup vs baseline: 8.9829x; 1.0023x over previous
import functools

import jax
import jax.numpy as jnp
from jax import lax
from jax.experimental import pallas as pl
from jax.experimental.pallas import tpu as pltpu
from jax.experimental.pallas import tpu_sc as plsc

_NUM_CORES = 2
_NUM_SUBCORES = 16
_NW = _NUM_CORES * _NUM_SUBCORES
_CHUNK = 2560
_LANES = 16
_OUT_W = 128


@functools.partial(jax.jit, static_argnames=("n_edges", "n_types", "dim"))
def _sc_embedding_lookup(table, edge_types, *, n_edges, n_types, dim):
    total_chunks = n_edges // _CHUNK
    assert total_chunks * _CHUNK == n_edges
    base_cnt = total_chunks // _NW
    rem = total_chunks % _NW
    max_cnt = base_cnt + (1 if rem else 0)
    max_cnt += max_cnt % 2
    out_rows = n_edges * dim // _OUT_W
    crows = _CHUNK * dim // _OUT_W
    lanes_per_row = _OUT_W // dim
    mesh = plsc.VectorSubcoreMesh(core_axis_name="c", subcore_axis_name="s")

    @functools.partial(
        pl.kernel,
        mesh=mesh,
        out_type=jax.ShapeDtypeStruct((1, out_rows, _OUT_W), jnp.float32),
        scratch_types=[
            pltpu.VMEM((n_types * dim,), jnp.float32),
            pltpu.VMEM((_CHUNK,), jnp.int32),
            pltpu.VMEM((_CHUNK,), jnp.int32),
            pltpu.VMEM((crows, _OUT_W), jnp.float32),
            pltpu.VMEM((crows, _OUT_W), jnp.float32),
            pltpu.SemaphoreType.DMA,
            pltpu.SemaphoreType.DMA,
            pltpu.SemaphoreType.DMA,
        ],
        compiler_params=pltpu.CompilerParams(needs_layout_passes=False),
    )
    def k(table_hbm, idx_hbm, out_hbm, table_v, idx0, idx1, rows0, rows1,
          isem, ssem0, ssem1):
        wid = lax.axis_index("s") * _NUM_CORES + lax.axis_index("c")
        n_w = base_cnt + jnp.where(wid < rem, 1, 0)
        start_w = wid * base_cnt + jnp.minimum(wid, rem)
        idx_bufs = (idx0, idx1)
        rows_bufs = (rows0, rows1)
        ssems = (ssem0, ssem1)

        pltpu.sync_copy(table_hbm, table_v)
        pltpu.async_copy(
            idx_hbm.at[0, pl.ds(start_w * _CHUNK, _CHUNK)], idx0, isem)

        def compute_chunk(idx_v, rows_v):
            def grp(g, _):
                e0 = g * _LANES
                ivec = idx_v[pl.ds(e0, _LANES)] * dim
                for u in range(_LANES):
                    r = (_LANES // lanes_per_row) * g + u // lanes_per_row
                    c = (u % lanes_per_row) * dim
                    rows_v[r, pl.ds(c, dim)] = table_v[pl.ds(ivec[u], dim)]
                return ()

            lax.fori_loop(0, _CHUNK // _LANES, grp, (), unroll=False)

        def outer(o, _):
            for b in range(2):
                i = o * 2 + b

                @pl.when(i < n_w)
                def _():
                    base = (start_w + i) * _CHUNK
                    row_base = (start_w + i) * crows
                    pltpu.make_async_copy(
                        idx_hbm.at[0, pl.ds(base, _CHUNK)], idx_bufs[b], isem
                    ).wait()

                    @pl.when(i + 1 < n_w)
                    def _():
                        pltpu.async_copy(
                            idx_hbm.at[0, pl.ds(base + _CHUNK, _CHUNK)],
                            idx_bufs[1 - b], isem)

                    @pl.when(i >= 2)
                    def _():
                        pltpu.make_async_copy(
                            rows_bufs[b],
                            out_hbm.at[0, pl.ds(row_base, crows)],
                            ssems[b]).wait()

                    compute_chunk(idx_bufs[b], rows_bufs[b])
                    pltpu.async_copy(
                        rows_bufs[b], out_hbm.at[0, pl.ds(row_base, crows)],
                        ssems[b])
            return ()

        lax.fori_loop(0, max_cnt // 2, outer, ())

        for b in range(2):
            pltpu.make_async_copy(
                rows_bufs[b], out_hbm.at[0, pl.ds(0, crows)],
                ssems[b]).wait()

    return k(table, edge_types)


def kernel(edge_types, table):
    batch, n_edges = edge_types.shape
    n_types, dim = table.shape
    out = _sc_embedding_lookup(
        table.reshape(-1), edge_types.astype(jnp.int32),
        n_edges=n_edges, n_types=n_types, dim=dim)
    return out.reshape(batch, n_edges, dim)


# speedup vs baseline: 150.4957x; 16.7536x over previous
import functools

import jax
import jax.numpy as jnp
from jax import lax
from jax.experimental import pallas as pl
from jax.experimental.pallas import tpu as pltpu
from jax.experimental.pallas import tpu_sc as plsc

_NUM_CORES = 2
_NUM_SUBCORES = 16
_NW = _NUM_CORES * _NUM_SUBCORES
_CHUNK = 2560
_LANES = 16
_GATHER_DNUMS = lax.GatherDimensionNumbers(
    offset_dims=(), collapsed_slice_dims=(0,), start_index_map=(0,))


@functools.partial(jax.jit, static_argnames=("n_edges", "dim"))
def _sc_embedding_lookup(table_t, edge_types, *, n_edges, dim):
    total_chunks = n_edges // _CHUNK
    assert total_chunks * _CHUNK == n_edges
    base_cnt = total_chunks // _NW
    rem = total_chunks % _NW
    max_cnt = base_cnt + (1 if rem else 0)
    max_cnt += max_cnt % 2
    mesh = plsc.VectorSubcoreMesh(core_axis_name="c", subcore_axis_name="s")

    @functools.partial(
        pl.kernel,
        mesh=mesh,
        out_type=jax.ShapeDtypeStruct((dim, n_edges), jnp.float32),
        scratch_types=[
            pltpu.VMEM((dim * _LANES,), jnp.float32),
            pltpu.VMEM((_CHUNK,), jnp.int32),
            pltpu.VMEM((_CHUNK,), jnp.int32),
            pltpu.VMEM((dim, _CHUNK), jnp.float32),
            pltpu.VMEM((dim, _CHUNK), jnp.float32),
            pltpu.SemaphoreType.DMA,
            pltpu.SemaphoreType.DMA,
            pltpu.SemaphoreType.DMA,
        ],
        compiler_params=pltpu.CompilerParams(needs_layout_passes=False),
    )
    def k(table_hbm, idx_hbm, out_hbm, table_v, idx0, idx1, rows0, rows1,
          isem, ssem0, ssem1):
        wid = lax.axis_index("s") * _NUM_CORES + lax.axis_index("c")
        n_w = base_cnt + jnp.where(wid < rem, 1, 0)
        start_w = wid * base_cnt + jnp.minimum(wid, rem)
        idx_bufs = (idx0, idx1)
        rows_bufs = (rows0, rows1)
        ssems = (ssem0, ssem1)

        pltpu.sync_copy(table_hbm, table_v)
        tcol = [table_v[pl.ds(d * _LANES, _LANES)] for d in range(dim)]
        pltpu.async_copy(
            idx_hbm.at[0, pl.ds(start_w * _CHUNK, _CHUNK)], idx0, isem)

        def compute_chunk(idx_v, rows_v):
            def grp(g, _):
                e0 = g * _LANES
                ivec = idx_v[pl.ds(e0, _LANES)][:, None]
                for d in range(dim):
                    rows_v[d, pl.ds(e0, _LANES)] = lax.gather(
                        tcol[d], ivec, _GATHER_DNUMS, (1,),
                        mode=lax.GatherScatterMode.PROMISE_IN_BOUNDS)
                return ()

            lax.fori_loop(0, _CHUNK // _LANES, grp, (), unroll=False)

        def outer(o, _):
            for b in range(2):
                i = o * 2 + b

                @pl.when(i < n_w)
                def _():
                    base = (start_w + i) * _CHUNK
                    pltpu.make_async_copy(
                        idx_hbm.at[0, pl.ds(base, _CHUNK)], idx_bufs[b], isem
                    ).wait()

                    @pl.when(i + 1 < n_w)
                    def _():
                        pltpu.async_copy(
                            idx_hbm.at[0, pl.ds(base + _CHUNK, _CHUNK)],
                            idx_bufs[1 - b], isem)

                    @pl.when(i >= 2)
                    def _():
                        pltpu.make_async_copy(
                            rows_bufs[b],
                            out_hbm.at[:, pl.ds(base, _CHUNK)],
                            ssems[b]).wait()

                    compute_chunk(idx_bufs[b], rows_bufs[b])
                    pltpu.async_copy(
                        rows_bufs[b], out_hbm.at[:, pl.ds(base, _CHUNK)],
                        ssems[b])
            return ()

        lax.fori_loop(0, max_cnt // 2, outer, ())

        for b in range(2):
            pltpu.make_async_copy(
                rows_bufs[b], out_hbm.at[:, pl.ds(0, _CHUNK)],
                ssems[b]).wait()

    return k(table_t, edge_types)


def kernel(edge_types, table):
    batch, n_edges = edge_types.shape
    n_types, dim = table.shape
    table_t = jnp.zeros((dim, _LANES), jnp.float32)
    table_t = table_t.at[:, :n_types].set(table.T).reshape(-1)
    out = _sc_embedding_lookup(
        table_t, edge_types.astype(jnp.int32), n_edges=n_edges, dim=dim)
    return out.T.reshape(batch, n_edges, dim)


# speedup vs baseline: 154.4178x; 1.0261x over previous
import functools

import jax
import jax.numpy as jnp
from jax import lax
from jax.experimental import pallas as pl
from jax.experimental.pallas import tpu as pltpu
from jax.experimental.pallas import tpu_sc as plsc

_NUM_CORES = 2
_NUM_SUBCORES = 16
_NW = _NUM_CORES * _NUM_SUBCORES
_CHUNK = 3200
_LANES = 16
_GATHER_DNUMS = lax.GatherDimensionNumbers(
    offset_dims=(), collapsed_slice_dims=(0,), start_index_map=(0,))


@functools.partial(jax.jit, static_argnames=("n_edges", "dim"))
def _sc_embedding_lookup(table_t, edge_types, *, n_edges, dim):
    total_chunks = n_edges // _CHUNK
    assert total_chunks * _CHUNK == n_edges
    base_cnt = total_chunks // _NW
    rem = total_chunks % _NW
    max_cnt = base_cnt + (1 if rem else 0)
    max_cnt += max_cnt % 2
    mesh = plsc.VectorSubcoreMesh(core_axis_name="c", subcore_axis_name="s")

    @functools.partial(
        pl.kernel,
        mesh=mesh,
        out_type=jax.ShapeDtypeStruct((dim, n_edges), jnp.float32),
        scratch_types=[
            pltpu.VMEM((dim * _LANES,), jnp.float32),
            pltpu.VMEM((_CHUNK,), jnp.int32),
            pltpu.VMEM((_CHUNK,), jnp.int32),
            pltpu.VMEM((dim, _CHUNK), jnp.float32),
            pltpu.VMEM((dim, _CHUNK), jnp.float32),
            pltpu.SemaphoreType.DMA,
            pltpu.SemaphoreType.DMA,
            pltpu.SemaphoreType.DMA,
        ],
        compiler_params=pltpu.CompilerParams(needs_layout_passes=False),
    )
    def k(table_hbm, idx_hbm, out_hbm, table_v, idx0, idx1, rows0, rows1,
          isem, ssem0, ssem1):
        wid = lax.axis_index("s") * _NUM_CORES + lax.axis_index("c")
        n_w = base_cnt + jnp.where(wid < rem, 1, 0)
        start_w = wid * base_cnt + jnp.minimum(wid, rem)
        idx_bufs = (idx0, idx1)
        rows_bufs = (rows0, rows1)
        ssems = (ssem0, ssem1)

        pltpu.sync_copy(table_hbm, table_v)
        tcol = [table_v[pl.ds(d * _LANES, _LANES)] for d in range(dim)]
        pltpu.async_copy(
            idx_hbm.at[0, pl.ds(start_w * _CHUNK, _CHUNK)], idx0, isem)

        def compute_chunk(idx_v, rows_v):
            def grp(g, _):
                e0 = g * _LANES
                ivec = idx_v[pl.ds(e0, _LANES)][:, None]
                for d in range(dim):
                    rows_v[d, pl.ds(e0, _LANES)] = lax.gather(
                        tcol[d], ivec, _GATHER_DNUMS, (1,),
                        mode=lax.GatherScatterMode.PROMISE_IN_BOUNDS)
                return ()

            lax.fori_loop(0, _CHUNK // _LANES, grp, (), unroll=2)

        def outer(o, _):
            for b in range(2):
                i = o * 2 + b

                @pl.when(i < n_w)
                def _():
                    base = (start_w + i) * _CHUNK
                    pltpu.make_async_copy(
                        idx_hbm.at[0, pl.ds(base, _CHUNK)], idx_bufs[b], isem
                    ).wait()

                    @pl.when(i + 1 < n_w)
                    def _():
                        pltpu.async_copy(
                            idx_hbm.at[0, pl.ds(base + _CHUNK, _CHUNK)],
                            idx_bufs[1 - b], isem)

                    @pl.when(i >= 2)
                    def _():
                        pltpu.make_async_copy(
                            rows_bufs[b],
                            out_hbm.at[:, pl.ds(base, _CHUNK)],
                            ssems[b]).wait()

                    compute_chunk(idx_bufs[b], rows_bufs[b])
                    pltpu.async_copy(
                        rows_bufs[b], out_hbm.at[:, pl.ds(base, _CHUNK)],
                        ssems[b])
            return ()

        lax.fori_loop(0, max_cnt // 2, outer, ())

        for b in range(2):
            pltpu.make_async_copy(
                rows_bufs[b], out_hbm.at[:, pl.ds(0, _CHUNK)],
                ssems[b]).wait()

    return k(table_t, edge_types)


def kernel(edge_types, table):
    batch, n_edges = edge_types.shape
    n_types, dim = table.shape
    table_t = jnp.zeros((dim, _LANES), jnp.float32)
    table_t = table_t.at[:, :n_types].set(table.T).reshape(-1)
    out = _sc_embedding_lookup(
        table_t, edge_types.astype(jnp.int32), n_edges=n_edges, dim=dim)
    return out.T.reshape(batch, n_edges, dim)
